# Initial kernel scaffold; baseline (speedup 1.0000x reference)
import jax
import jax.numpy as jnp
from jax import lax
from jax.experimental import pallas as pl
from jax.experimental.pallas import tpu as pltpu
from jax.experimental.pallas import tpu_sc as plsc

P = 32
C = 96
H = W = 16
HO = WO = 18
IN_PER_PATCH = C * H * W
OUT_PER_PATCH = C * HO * WO
NC, NS = 2, 16
NW = NC * NS


def _body(x_hbm, out_hbm, in_v, out_v):
    cid = lax.axis_index("c")
    sid = lax.axis_index("s")
    wid = sid * NC + cid
    patches_per_w = x_hbm.shape[0] // NW

    iota = lax.iota(jnp.int32, 16)
    corner_mask = (iota == 0) | (iota == 15)

    def patch_body(i, _):
        p = wid * patches_per_w + i
        gr = p // P
        gc = p % P
        st = jnp.where(gr == 0, 0.0, 0.5)
        sb = jnp.where(gr == P - 1, 0.0, 0.5)
        sl = jnp.where(gc == 0, 0.0, 0.5)
        sr = jnp.where(gc == P - 1, 0.0, 0.5)
        mt = jnp.where(gr == 0, 0.0, 1.0)
        mb = jnp.where(gr == P - 1, 0.0, 1.0)
        ml = jnp.where(gc == 0, 0.0, 1.0)
        mr = jnp.where(gc == P - 1, 0.0, 1.0)
        ctop = jnp.where(iota == 0, mt * ml, mt * mr)
        cbot = jnp.where(iota == 0, mb * ml, mb * mr)

        pltpu.sync_copy(x_hbm.at[p], in_v)

        def chan_body(c, _):
            ib = c * (H * W)
            ob = c * (HO * WO)
            rows = [in_v[pl.ds(ib + r * W, 16)] for r in range(H)]
            for r in range(H):
                plsc.store_scatter(out_v, [ob + (r + 1) * WO + 1 + iota], rows[r])
            top = (rows[0] + rows[1]) * st
            plsc.store_scatter(out_v, [ob + 1 + iota], top)
            bot = (rows[H - 2] + rows[H - 1]) * sb
            plsc.store_scatter(out_v, [ob + (HO - 1) * WO + 1 + iota], bot)
            col_idx = ib + iota * W
            c0 = plsc.load_gather(in_v, [col_idx])
            c1 = plsc.load_gather(in_v, [col_idx + 1])
            lft = (c0 + c1) * sl
            plsc.store_scatter(out_v, [ob + WO + WO * iota], lft)
            c14 = plsc.load_gather(in_v, [col_idx + (W - 2)])
            c15 = plsc.load_gather(in_v, [col_idx + (W - 1)])
            rgt = (c14 + c15) * sr
            plsc.store_scatter(out_v, [ob + WO + (WO - 1) + WO * iota], rgt)
            plsc.store_scatter(
                out_v,
                [jnp.where(iota == 0, ob, ob + WO - 1)],
                rows[0] * ctop,
                mask=corner_mask,
            )
            plsc.store_scatter(
                out_v,
                [jnp.where(iota == 0, ob + (HO - 1) * WO,
                           ob + (HO - 1) * WO + WO - 1)],
                rows[H - 1] * cbot,
                mask=corner_mask,
            )
            return 0

        lax.fori_loop(0, C, chan_body, 0)
        pltpu.sync_copy(out_v, out_hbm.at[p])
        return 0

    lax.fori_loop(0, patches_per_w, patch_body, 0)


def kernel(x):
    b = x.shape[0]
    x2 = x.reshape(b, IN_PER_PATCH)
    mesh = plsc.VectorSubcoreMesh(
        core_axis_name="c", subcore_axis_name="s",
        num_cores=NC, num_subcores=NS,
    )
    out = pl.kernel(
        _body,
        out_type=jax.ShapeDtypeStruct((b, OUT_PER_PATCH), jnp.float32),
        mesh=mesh,
        scratch_types=[
            pltpu.VMEM((IN_PER_PATCH,), jnp.float32),
            pltpu.VMEM((OUT_PER_PATCH,), jnp.float32),
        ],
    )(x2)
    return out.reshape(b, C, HO, WO)


# speedup vs baseline: 2.3323x; 2.3323x over previous
import jax
import jax.numpy as jnp
from jax import lax
from jax.experimental import pallas as pl
from jax.experimental.pallas import tpu as pltpu
from jax.experimental.pallas import tpu_sc as plsc

P = 32
C = 96
H = W = 16
HO = WO = 18
IN_PER_PATCH = C * H * W
OUT_PER_PATCH = C * HO * WO
NC, NS = 2, 16
NW = NC * NS


def _body(x_hbm, out_hbm, in_v, out_v):
    cid = lax.axis_index("c")
    sid = lax.axis_index("s")
    wid = sid * NC + cid
    patches_per_w = x_hbm.shape[0] // NW

    iota = lax.iota(jnp.int32, 16)
    corner_mask = (iota == 0) | (iota == 15)

    def patch_body(i, _):
        p = wid * patches_per_w + i
        gr = p // P
        gc = p % P
        st = jnp.where(gr == 0, 0.0, 0.5)
        sb = jnp.where(gr == P - 1, 0.0, 0.5)
        sl = jnp.where(gc == 0, 0.0, 0.5)
        sr = jnp.where(gc == P - 1, 0.0, 0.5)
        mt = jnp.where(gr == 0, 0.0, 1.0)
        mb = jnp.where(gr == P - 1, 0.0, 1.0)
        ml = jnp.where(gc == 0, 0.0, 1.0)
        mr = jnp.where(gc == P - 1, 0.0, 1.0)
        ctop = jnp.where(iota == 0, mt * ml, mt * mr)
        cbot = jnp.where(iota == 0, mb * ml, mb * mr)

        pltpu.sync_copy(x_hbm.at[p], in_v)

        def chan_body(c, _):
            ib = c * (H * W)
            ob = c * (HO * WO)
            rows = [in_v[pl.ds(ib + r * W, 16)] for r in range(H)]
            for r in range(H):
                plsc.store_scatter(out_v, [ob + (r + 1) * WO + 1 + iota], rows[r])
            top = (rows[0] + rows[1]) * st
            plsc.store_scatter(out_v, [ob + 1 + iota], top)
            bot = (rows[H - 2] + rows[H - 1]) * sb
            plsc.store_scatter(out_v, [ob + (HO - 1) * WO + 1 + iota], bot)
            col_idx = ib + iota * W
            c0 = plsc.load_gather(in_v, [col_idx])
            c1 = plsc.load_gather(in_v, [col_idx + 1])
            lft = (c0 + c1) * sl
            plsc.store_scatter(out_v, [ob + WO + WO * iota], lft)
            c14 = plsc.load_gather(in_v, [col_idx + (W - 2)])
            c15 = plsc.load_gather(in_v, [col_idx + (W - 1)])
            rgt = (c14 + c15) * sr
            plsc.store_scatter(out_v, [ob + WO + (WO - 1) + WO * iota], rgt)
            plsc.store_scatter(
                out_v,
                [jnp.where(iota == 0, ob, ob + WO - 1)],
                rows[0] * ctop,
                mask=corner_mask,
            )
            plsc.store_scatter(
                out_v,
                [jnp.where(iota == 0, ob + (HO - 1) * WO,
                           ob + (HO - 1) * WO + WO - 1)],
                rows[H - 1] * cbot,
                mask=corner_mask,
            )
            return 0

        lax.fori_loop(0, C, chan_body, 0)
        pltpu.sync_copy(out_v, out_hbm.at[p])
        return 0

    lax.fori_loop(0, patches_per_w, patch_body, 0)


def kernel(x):
    b = x.shape[0]
    x2 = x.reshape(b, IN_PER_PATCH)
    mesh = plsc.VectorSubcoreMesh(
        core_axis_name="c", subcore_axis_name="s",
        num_cores=NC, num_subcores=NS,
    )
    out = pl.kernel(
        _body,
        out_type=jax.ShapeDtypeStruct((b, OUT_PER_PATCH), jnp.float32),
        mesh=mesh,
        compiler_params=pltpu.CompilerParams(needs_layout_passes=False),
        scratch_types=[
            pltpu.VMEM((IN_PER_PATCH,), jnp.float32),
            pltpu.VMEM((OUT_PER_PATCH,), jnp.float32),
        ],
    )(x2)
    return out.reshape(b, C, HO, WO)
